# baseline (speedup 1.0000x reference)
import functools

import jax
import jax.numpy as jnp
from jax import lax
from jax.experimental import pallas as pl
from jax.experimental.pallas import tpu as pltpu
from jax.experimental.pallas import tpu_sc as plsc

N_NODES = 10000
N_EDGES = 320000
D_FEAT = 128
OUT_DIM = 256

NC = 2
NS = 16
DHALF = D_FEAT // NC
E_PER_S = N_EDGES // NS
CHUNK = 128
NCHUNK = -(-E_PER_S // CHUNK)
E_PAD_S = NCHUNK * CHUNK
ROWS_PER_TILE = N_NODES // NS
LANES = 16
NBUF = 4


def _sc_aggregate(xv, col3, row3, val3, zeros):
  mesh = plsc.VectorSubcoreMesh(core_axis_name="c", subcore_axis_name="s")

  @functools.partial(
      pl.kernel,
      out_type=jax.ShapeDtypeStruct((NC, NS, ROWS_PER_TILE, DHALF),
                                    jnp.float32),
      mesh=mesh,
      scratch_types=[
          pltpu.VMEM((NCHUNK, CHUNK), jnp.int32),
          pltpu.VMEM((NCHUNK, CHUNK), jnp.int32),
          pltpu.VMEM((NBUF, CHUNK), jnp.float32),
          pltpu.VMEM((NBUF, CHUNK, DHALF), jnp.float32),
          pltpu.VMEM((2, CHUNK, DHALF), jnp.float32),
          pltpu.VMEM_SHARED((N_NODES, DHALF), jnp.float32),
          pltpu.SemaphoreType.DMA((NBUF,)),
          pltpu.SemaphoreType.DMA((NBUF,)),
          pltpu.SemaphoreType.DMA,
      ],
      compiler_params=pltpu.CompilerParams(use_tc_tiling_on_sc=False),
  )
  def agg(x_hbm, col_hbm, row_hbm, val_hbm, z_hbm, out_hbm,
          col_v, row_v, vbuf, gbuf, sbuf, acc, gsem, vsem, ssem):
    c = lax.axis_index("c")
    s = lax.axis_index("s")

    pltpu.sync_copy(z_hbm,
                    acc.at[pl.ds(s * ROWS_PER_TILE, ROWS_PER_TILE)])
    pltpu.sync_copy(col_hbm.at[s], col_v)
    pltpu.sync_copy(row_hbm.at[s], row_v)

    @plsc.parallel_loop(0, NCHUNK, unroll=2)
    def _(k):
      for g in range(CHUNK // LANES):
        sl = pl.ds(g * LANES, LANES)
        col_v[k, sl] = col_v[k, sl] * 2 + c

    plsc.subcore_barrier()

    vs = val_hbm.at[s]

    def start_fetch(k, b):
      pltpu.async_copy(x_hbm.at[col_v.at[k]], gbuf.at[b], gsem.at[b])
      pltpu.async_copy(vs.at[k], vbuf.at[b], vsem.at[b])

    def wait_fetch(k, b):
      pltpu.make_async_copy(x_hbm.at[col_v.at[k]], gbuf.at[b],
                            gsem.at[b]).wait()
      pltpu.make_async_copy(vs.at[k], vbuf.at[b], vsem.at[b]).wait()

    def scale_chunk(k, b, sb):
      @plsc.parallel_loop(0, CHUNK // LANES, unroll=4)
      def _(g):
        vv = vbuf[b, pl.ds(g * LANES, LANES)]
        for e16 in range(LANES):
          v = vv[e16]
          e = g * LANES + e16
          for j in range(DHALF // LANES):
            sl = pl.ds(j * LANES, LANES)
            sbuf[sb, e, sl] = gbuf[b, e, sl] * v

    def start_scatter(k, sb):
      pltpu.async_copy(sbuf.at[sb], acc.at[row_v.at[k]], ssem, add=True)

    def wait_one_scatter():
      pltpu.make_async_copy(sbuf.at[0], acc.at[row_v.at[0]], ssem).wait()

    for i in range(NBUF - 1):
      start_fetch(i, i)

    wait_fetch(0, 0)
    start_fetch(NBUF - 1, NBUF - 1)
    scale_chunk(0, 0, 0)
    start_scatter(0, 0)

    def chunk_body(k, carry):
      b = lax.rem(k, NBUF)
      bn = lax.rem(k + NBUF - 1, NBUF)
      sb = lax.rem(k, 2)
      wait_fetch(k, b)
      start_fetch(k + NBUF - 1, bn)
      scale_chunk(k, b, sb)
      start_scatter(k, sb)
      wait_one_scatter()
      return carry

    lax.fori_loop(1, NCHUNK - (NBUF - 1), chunk_body, 0)

    def tail_body(k, carry):
      b = lax.rem(k, NBUF)
      sb = lax.rem(k, 2)
      wait_fetch(k, b)
      scale_chunk(k, b, sb)
      start_scatter(k, sb)
      wait_one_scatter()
      return carry

    lax.fori_loop(NCHUNK - (NBUF - 1), NCHUNK, tail_body, 0)
    wait_one_scatter()

    plsc.subcore_barrier()
    pltpu.sync_copy(acc.at[pl.ds(s * ROWS_PER_TILE, ROWS_PER_TILE)],
                    out_hbm.at[c, s])

  return agg(xv, col3, row3, val3, zeros)


def _mm_body(p0_ref, p1_ref, w_ref, o_ref):
  o_ref[...] = (
      jnp.dot(p0_ref[0], w_ref[:DHALF], preferred_element_type=jnp.float32)
      + jnp.dot(p1_ref[0], w_ref[DHALF:], preferred_element_type=jnp.float32))


def _tc_matmul(p, w):
  bm = 1000
  return pl.pallas_call(
      _mm_body,
      grid=(N_NODES // bm,),
      in_specs=[
          pl.BlockSpec((1, bm, DHALF), lambda i: (0, i, 0)),
          pl.BlockSpec((1, bm, DHALF), lambda i: (1, i, 0)),
          pl.BlockSpec((D_FEAT, OUT_DIM), lambda i: (0, 0)),
      ],
      out_specs=pl.BlockSpec((bm, OUT_DIM), lambda i: (i, 0)),
      out_shape=jax.ShapeDtypeStruct((N_NODES, OUT_DIM), jnp.float32),
  )(p, p, w)


def kernel(x, adj_row, adj_col, adj_values, kernel):
  xv = x.reshape(NC * N_NODES, DHALF)
  pad = ((0, 0), (0, E_PAD_S - E_PER_S))
  col3 = jnp.pad(adj_col.reshape(NS, E_PER_S), pad).reshape(NS, NCHUNK, CHUNK)
  row3 = jnp.pad(adj_row.reshape(NS, E_PER_S), pad).reshape(NS, NCHUNK, CHUNK)
  val3 = jnp.pad(adj_values.reshape(NS, E_PER_S), pad).reshape(
      NS, NCHUNK, CHUNK)
  zeros = jnp.zeros((ROWS_PER_TILE, DHALF), jnp.float32)
  parts = _sc_aggregate(xv, col3, row3, val3, zeros)
  p = parts.reshape(NC, N_NODES, DHALF)
  return _tc_matmul(p, kernel)


# speedup vs baseline: 1.0069x; 1.0069x over previous
import functools

import jax
import jax.numpy as jnp
from jax import lax
from jax.experimental import pallas as pl
from jax.experimental.pallas import tpu as pltpu
from jax.experimental.pallas import tpu_sc as plsc

N_NODES = 10000
N_EDGES = 320000
D_FEAT = 128
OUT_DIM = 256

NC = 2
NS = 16
DHALF = D_FEAT // NC
E_PER_S = N_EDGES // NS
CHUNK = 128
NCHUNK = -(-E_PER_S // CHUNK)
E_PAD_S = NCHUNK * CHUNK
ROWS_PER_TILE = N_NODES // NS
LANES = 16
NBUF = 4


def _sc_aggregate(xv, col3, row3, val3, zeros):
  mesh = plsc.VectorSubcoreMesh(core_axis_name="c", subcore_axis_name="s")

  @functools.partial(
      pl.kernel,
      out_type=jax.ShapeDtypeStruct((NC, NS, ROWS_PER_TILE, DHALF),
                                    jnp.float32),
      mesh=mesh,
      scratch_types=[
          pltpu.VMEM((NCHUNK, CHUNK), jnp.int32),
          pltpu.VMEM((NCHUNK, CHUNK), jnp.int32),
          pltpu.VMEM((NBUF, CHUNK), jnp.float32),
          pltpu.VMEM((NBUF, CHUNK, DHALF), jnp.float32),
          pltpu.VMEM((2, CHUNK, DHALF), jnp.float32),
          pltpu.VMEM_SHARED((N_NODES, DHALF), jnp.float32),
          pltpu.SemaphoreType.DMA((NBUF,)),
          pltpu.SemaphoreType.DMA((NBUF,)),
          pltpu.SemaphoreType.DMA((2,)),
      ],
      compiler_params=pltpu.CompilerParams(use_tc_tiling_on_sc=False),
  )
  def agg(x_hbm, col_hbm, row_hbm, val_hbm, z_hbm, out_hbm,
          col_v, row_v, vbuf, gbuf, sbuf, acc, gsem, vsem, ssem):
    c = lax.axis_index("c")
    s = lax.axis_index("s")

    pltpu.sync_copy(z_hbm,
                    acc.at[pl.ds(s * ROWS_PER_TILE, ROWS_PER_TILE)])
    pltpu.sync_copy(col_hbm.at[s], col_v)
    pltpu.sync_copy(row_hbm.at[s], row_v)

    @plsc.parallel_loop(0, NCHUNK, unroll=2)
    def _(k):
      for g in range(CHUNK // LANES):
        sl = pl.ds(g * LANES, LANES)
        col_v[k, sl] = col_v[k, sl] * 2 + c

    plsc.subcore_barrier()

    vs = val_hbm.at[s]

    def start_fetch(k, b):
      pltpu.async_copy(x_hbm.at[col_v.at[k]], gbuf.at[b], gsem.at[b])
      pltpu.async_copy(vs.at[k], vbuf.at[b], vsem.at[b])

    def wait_fetch(k, b):
      pltpu.make_async_copy(x_hbm.at[col_v.at[k]], gbuf.at[b],
                            gsem.at[b]).wait()
      pltpu.make_async_copy(vs.at[k], vbuf.at[b], vsem.at[b]).wait()

    def scale_chunk(k, b, sb):
      @plsc.parallel_loop(0, CHUNK // LANES, unroll=2)
      def _(g):
        vv = vbuf[b, pl.ds(g * LANES, LANES)]
        for e16 in range(LANES):
          v = vv[e16]
          e = g * LANES + e16
          for j in range(DHALF // LANES):
            sl = pl.ds(j * LANES, LANES)
            sbuf[sb, e, sl] = gbuf[b, e, sl] * v

    def start_scatter(k, sb):
      pltpu.async_copy(sbuf.at[sb], acc.at[row_v.at[k]], ssem.at[sb],
                       add=True)

    def wait_one_scatter(sb):
      pltpu.make_async_copy(sbuf.at[sb], acc.at[row_v.at[0]],
                            ssem.at[sb]).wait()

    for i in range(NBUF - 1):
      start_fetch(i, i)

    wait_fetch(0, 0)
    start_fetch(NBUF - 1, NBUF - 1)
    scale_chunk(0, 0, 0)
    start_scatter(0, 0)

    wait_fetch(1, 1)
    start_fetch(NBUF, 0)
    scale_chunk(1, 1, 1)
    start_scatter(1, 1)

    def chunk_body(k, carry):
      b = lax.rem(k, NBUF)
      bn = lax.rem(k + NBUF - 1, NBUF)
      sb = lax.rem(k, 2)
      wait_fetch(k, b)
      start_fetch(k + NBUF - 1, bn)
      wait_one_scatter(sb)
      scale_chunk(k, b, sb)
      start_scatter(k, sb)
      return carry

    lax.fori_loop(2, NCHUNK - (NBUF - 1), chunk_body, 0)

    def tail_body(k, carry):
      b = lax.rem(k, NBUF)
      sb = lax.rem(k, 2)
      wait_fetch(k, b)
      wait_one_scatter(sb)
      scale_chunk(k, b, sb)
      start_scatter(k, sb)
      return carry

    lax.fori_loop(NCHUNK - (NBUF - 1), NCHUNK, tail_body, 0)
    wait_one_scatter(0)
    wait_one_scatter(1)

    plsc.subcore_barrier()
    pltpu.sync_copy(acc.at[pl.ds(s * ROWS_PER_TILE, ROWS_PER_TILE)],
                    out_hbm.at[c, s])

  return agg(xv, col3, row3, val3, zeros)


def _mm_body(p0_ref, p1_ref, w_ref, o_ref):
  o_ref[...] = (
      jnp.dot(p0_ref[0], w_ref[:DHALF], preferred_element_type=jnp.float32)
      + jnp.dot(p1_ref[0], w_ref[DHALF:], preferred_element_type=jnp.float32))


def _tc_matmul(p, w):
  bm = 1000
  return pl.pallas_call(
      _mm_body,
      grid=(N_NODES // bm,),
      in_specs=[
          pl.BlockSpec((1, bm, DHALF), lambda i: (0, i, 0)),
          pl.BlockSpec((1, bm, DHALF), lambda i: (1, i, 0)),
          pl.BlockSpec((D_FEAT, OUT_DIM), lambda i: (0, 0)),
      ],
      out_specs=pl.BlockSpec((bm, OUT_DIM), lambda i: (i, 0)),
      out_shape=jax.ShapeDtypeStruct((N_NODES, OUT_DIM), jnp.float32),
  )(p, p, w)


def kernel(x, adj_row, adj_col, adj_values, kernel):
  xv = x.reshape(NC * N_NODES, DHALF)
  pad = ((0, 0), (0, E_PAD_S - E_PER_S))
  col3 = jnp.pad(adj_col.reshape(NS, E_PER_S), pad).reshape(NS, NCHUNK, CHUNK)
  row3 = jnp.pad(adj_row.reshape(NS, E_PER_S), pad).reshape(NS, NCHUNK, CHUNK)
  val3 = jnp.pad(adj_values.reshape(NS, E_PER_S), pad).reshape(
      NS, NCHUNK, CHUNK)
  zeros = jnp.zeros((ROWS_PER_TILE, DHALF), jnp.float32)
  parts = _sc_aggregate(xv, col3, row3, val3, zeros)
  p = parts.reshape(NC, N_NODES, DHALF)
  return _tc_matmul(p, kernel)


# speedup vs baseline: 1.2527x; 1.2440x over previous
import functools

import jax
import jax.numpy as jnp
from jax import lax
from jax.experimental import pallas as pl
from jax.experimental.pallas import tpu as pltpu
from jax.experimental.pallas import tpu_sc as plsc

N_NODES = 10000
N_EDGES = 320000
D_FEAT = 128
OUT_DIM = 256

NC = 2
NS = 16
DHALF = D_FEAT // NC
E_PER_S = N_EDGES // NS
CHUNK = 128
NCHUNK = -(-E_PER_S // CHUNK)
E_PAD_S = NCHUNK * CHUNK
ROWS_PER_TILE = N_NODES // NS
LANES = 16
NBUF = 4


def _sc_aggregate(xv, col3, row3, val3, zeros):
  mesh = plsc.VectorSubcoreMesh(core_axis_name="c", subcore_axis_name="s")

  @functools.partial(
      pl.kernel,
      out_type=jax.ShapeDtypeStruct((NC, NS, ROWS_PER_TILE, DHALF),
                                    jnp.float32),
      mesh=mesh,
      scratch_types=[
          pltpu.VMEM((NCHUNK, CHUNK), jnp.int32),
          pltpu.VMEM((NCHUNK, CHUNK), jnp.int32),
          pltpu.VMEM((NBUF, CHUNK), jnp.float32),
          pltpu.VMEM((NBUF, CHUNK, DHALF), jnp.bfloat16),
          pltpu.VMEM((2, CHUNK, DHALF), jnp.float32),
          pltpu.VMEM_SHARED((N_NODES, DHALF), jnp.float32),
          pltpu.SemaphoreType.DMA((NBUF,)),
          pltpu.SemaphoreType.DMA((NBUF,)),
          pltpu.SemaphoreType.DMA((2,)),
      ],
      compiler_params=pltpu.CompilerParams(use_tc_tiling_on_sc=False,
                                           needs_layout_passes=False),
  )
  def agg(x_hbm, col_hbm, row_hbm, val_hbm, z_hbm, out_hbm,
          col_v, row_v, vbuf, gbuf, sbuf, acc, gsem, vsem, ssem):
    c = lax.axis_index("c")
    s = lax.axis_index("s")

    pltpu.sync_copy(z_hbm,
                    acc.at[pl.ds(s * ROWS_PER_TILE, ROWS_PER_TILE)])
    pltpu.sync_copy(col_hbm.at[s], col_v)
    pltpu.sync_copy(row_hbm.at[s], row_v)

    @plsc.parallel_loop(0, NCHUNK, unroll=2)
    def _(k):
      for g in range(CHUNK // LANES):
        sl = pl.ds(g * LANES, LANES)
        col_v[k, sl] = col_v[k, sl] * 2 + c

    plsc.subcore_barrier()

    vs = val_hbm.at[s]

    def start_fetch(k, b):
      pltpu.async_copy(x_hbm.at[col_v.at[k]], gbuf.at[b], gsem.at[b])
      pltpu.async_copy(vs.at[k], vbuf.at[b], vsem.at[b])

    def wait_fetch(k, b):
      pltpu.make_async_copy(x_hbm.at[col_v.at[k]], gbuf.at[b],
                            gsem.at[b]).wait()
      pltpu.make_async_copy(vs.at[k], vbuf.at[b], vsem.at[b]).wait()

    def scale_chunk(k, b, sb):
      @plsc.parallel_loop(0, CHUNK // LANES, unroll=2)
      def _(g):
        vv = vbuf[b, pl.ds(g * LANES, LANES)]
        for e16 in range(LANES):
          v = vv[e16]
          e = g * LANES + e16
          for j in range(DHALF // (2 * LANES)):
            ab = gbuf[b, e, pl.ds(j * 2 * LANES, 2 * LANES)]
            a0, a1 = plsc.unpack(ab, format=plsc.PackFormat.INTERLEAVED)
            sbuf[sb, e, pl.ds(j * 2 * LANES, LANES)] = a0 * v
            sbuf[sb, e, pl.ds(j * 2 * LANES + LANES, LANES)] = a1 * v

    def start_scatter(k, sb):
      pltpu.async_copy(sbuf.at[sb], acc.at[row_v.at[k]], ssem.at[sb],
                       add=True)

    def wait_one_scatter(sb):
      pltpu.make_async_copy(sbuf.at[sb], acc.at[row_v.at[0]],
                            ssem.at[sb]).wait()

    for i in range(NBUF - 1):
      start_fetch(i, i)

    wait_fetch(0, 0)
    start_fetch(NBUF - 1, NBUF - 1)
    scale_chunk(0, 0, 0)
    start_scatter(0, 0)

    wait_fetch(1, 1)
    start_fetch(NBUF, 0)
    scale_chunk(1, 1, 1)
    start_scatter(1, 1)

    def chunk_body(k, carry):
      b = lax.rem(k, NBUF)
      bn = lax.rem(k + NBUF - 1, NBUF)
      sb = lax.rem(k, 2)
      wait_fetch(k, b)
      start_fetch(k + NBUF - 1, bn)
      wait_one_scatter(sb)
      scale_chunk(k, b, sb)
      start_scatter(k, sb)
      return carry

    lax.fori_loop(2, NCHUNK - (NBUF - 1), chunk_body, 0)

    def tail_body(k, carry):
      b = lax.rem(k, NBUF)
      sb = lax.rem(k, 2)
      wait_fetch(k, b)
      wait_one_scatter(sb)
      scale_chunk(k, b, sb)
      start_scatter(k, sb)
      return carry

    lax.fori_loop(NCHUNK - (NBUF - 1), NCHUNK, tail_body, 0)
    wait_one_scatter(0)
    wait_one_scatter(1)

    plsc.subcore_barrier()
    pltpu.sync_copy(acc.at[pl.ds(s * ROWS_PER_TILE, ROWS_PER_TILE)],
                    out_hbm.at[c, s])

  return agg(xv, col3, row3, val3, zeros)


def _mm_body(p0_ref, p1_ref, w_ref, o_ref):
  o_ref[...] = (
      jnp.dot(p0_ref[0], w_ref[:DHALF], preferred_element_type=jnp.float32)
      + jnp.dot(p1_ref[0], w_ref[DHALF:], preferred_element_type=jnp.float32))


def _tc_matmul(p, w):
  bm = 1000
  return pl.pallas_call(
      _mm_body,
      grid=(N_NODES // bm,),
      in_specs=[
          pl.BlockSpec((1, bm, DHALF), lambda i: (0, i, 0)),
          pl.BlockSpec((1, bm, DHALF), lambda i: (1, i, 0)),
          pl.BlockSpec((D_FEAT, OUT_DIM), lambda i: (0, 0)),
      ],
      out_specs=pl.BlockSpec((bm, OUT_DIM), lambda i: (i, 0)),
      out_shape=jax.ShapeDtypeStruct((N_NODES, OUT_DIM), jnp.float32),
  )(p, p, w)


def kernel(x, adj_row, adj_col, adj_values, kernel):
  xv = x.astype(jnp.bfloat16).reshape(NC * N_NODES, DHALF)
  g32 = jnp.arange(0, 32, 2, dtype=jnp.int32)
  perm64 = jnp.concatenate([g32, g32 + 1, 32 + g32, 32 + g32 + 1])
  wperm = jnp.concatenate([kernel[perm64], kernel[64 + perm64]], axis=0)
  pad = ((0, 0), (0, E_PAD_S - E_PER_S))
  col3 = jnp.pad(adj_col.reshape(NS, E_PER_S), pad).reshape(NS, NCHUNK, CHUNK)
  row3 = jnp.pad(adj_row.reshape(NS, E_PER_S), pad).reshape(NS, NCHUNK, CHUNK)
  val3 = jnp.pad(adj_values.reshape(NS, E_PER_S), pad).reshape(
      NS, NCHUNK, CHUNK)
  zeros = jnp.zeros((ROWS_PER_TILE, DHALF), jnp.float32)
  parts = _sc_aggregate(xv, col3, row3, val3, zeros)
  p = parts.reshape(NC, N_NODES, DHALF)
  return _tc_matmul(p, wperm)
